# baseline (speedup 1.0000x reference)
import functools

import jax
import jax.numpy as jnp
from jax.experimental import pallas as pl
from jax.experimental.pallas import tpu as pltpu

_J = 2


def _upsample_nn(r, fh, fw):
    Hc, Wc, C = r.shape
    r = jnp.broadcast_to(r[:, None, :, :], (Hc, fh, Wc, C))
    r = r.reshape(Hc * fh, Wc, C)
    r = jnp.broadcast_to(r[:, :, None, :], (Hc * fh, Wc, fw, C))
    return r.reshape(Hc * fh, Wc * fw, C)


def _mask_edge_rows(s, first, last):
    Hs = s.shape[0]
    ii = jax.lax.broadcasted_iota(jnp.int32, (Hs, 1, 1), 0)
    keep = ((ii > 0) | jnp.logical_not(first)) & (
        (ii < Hs - 1) | jnp.logical_not(last))
    return jnp.where(keep, s, jnp.zeros((), s.dtype))


def _strip(ref, B, j, first, last):
    H = ref.shape[0]
    t = jnp.maximum(B * j - 1, 0)
    bm = jnp.minimum(B * j + B, H - 1)
    s = jnp.concatenate(
        [ref[pl.ds(t, 1)], ref[pl.ds(B * j, B)], ref[pl.ds(bm, 1)]], axis=0)
    return _mask_edge_rows(s, first, last)


def _conv3x3_valid(x, w3, b):
    Hs, W, C = x.shape
    Co = w3.shape[-1]
    zcol = jnp.zeros((Hs, 1, C), x.dtype)
    p0 = jnp.concatenate([zcol, x[:, : W - 1, :]], axis=1)
    p2 = jnp.concatenate([x[:, 1:, :], zcol], axis=1)
    patch = jnp.concatenate([p0, x, p2], axis=-1).reshape(Hs * W, 3 * C)
    y0 = jnp.dot(patch, w3[0], preferred_element_type=jnp.float32)
    y1 = jnp.dot(patch, w3[1], preferred_element_type=jnp.float32)
    y2 = jnp.dot(patch, w3[2], preferred_element_type=jnp.float32)
    y0 = y0.reshape(Hs, W, Co)
    y1 = y1.reshape(Hs, W, Co)
    y2 = y2.reshape(Hs, W, Co)
    return (y0[: Hs - 2] + y1[1 : Hs - 1] + y2[2:]) + b.reshape(1, 1, Co)


def _fused_kernel(c3_ref, c4_ref, c5_ref,
                  w51_ref, b5_ref, w52_ref, b52_ref,
                  w41_ref, b4_ref, w42_ref, b42_ref,
                  w31_ref, b3_ref, w32_ref, b32_ref,
                  o3_ref, o4_ref, o5_ref,
                  p5x_s, p4x_s):
    H5, W5, C5c = c5_ref.shape[1:]
    H4, W4, C4c = c4_ref.shape[1:]
    H3, W3, C3c = c3_ref.shape[1:]
    Ch = w51_ref.shape[1]
    B5, B4, B3 = H5 // _J, H4 // _J, H3 // _J

    bf = jnp.bfloat16
    j = pl.program_id(1)
    first = j == 0
    last = j == _J - 1

    w51 = w51_ref[...].astype(bf)
    w41 = w41_ref[...].astype(bf)
    w31 = w31_ref[...].astype(bf)
    w52 = w52_ref[...].astype(bf)
    w42 = w42_ref[...].astype(bf)
    w32 = w32_ref[...].astype(bf)

    @pl.when(first)
    def _():
        x5 = c5_ref[0].reshape(H5 * W5, C5c).astype(bf)
        y5 = jnp.dot(x5, w51, preferred_element_type=jnp.float32)
        p5 = (y5 + b5_ref[...]).astype(bf).reshape(H5, W5, Ch)
        p5x_s[...] = p5
        x4 = c4_ref[0].reshape(H4 * W4, C4c).astype(bf)
        y4 = jnp.dot(x4, w41, preferred_element_type=jnp.float32)
        y4 = (y4 + b4_ref[...]).reshape(H4, W4, Ch)
        r4 = _upsample_nn(p5.astype(jnp.float32), H4 // H5, W4 // W5)
        p4x_s[...] = (y4 + r4).astype(bf)

    s5 = _strip(p5x_s, B5, j, first, last)
    o5_ref[...] = _conv3x3_valid(s5, w52, b52_ref[...])[None]
    s4 = _strip(p4x_s, B4, j, first, last)
    o4_ref[...] = _conv3x3_valid(s4, w42, b42_ref[...])[None]

    fh, fw = H3 // H4, W3 // W4
    t3 = jnp.maximum(B3 * j - 1, 0)
    bt3 = jnp.minimum(B3 * j + B3, H3 - 1)
    x3 = jnp.concatenate(
        [c3_ref[0, pl.ds(t3, 1)],
         c3_ref[0, pl.ds(B3 * j, B3)],
         c3_ref[0, pl.ds(bt3, 1)]], axis=0).astype(bf)
    y3 = jnp.dot(x3.reshape((B3 + 2) * W3, C3c), w31,
                 preferred_element_type=jnp.float32)
    y3 = (y3 + b3_ref[...]).reshape(B3 + 2, W3, Ch)
    rt = _upsample_nn(p4x_s[pl.ds(t3 // fh, 1)], 1, fw)
    rm = _upsample_nn(p4x_s[pl.ds(B4 * j, B4)], fh, fw)
    rb = _upsample_nn(p4x_s[pl.ds(bt3 // fh, 1)], 1, fw)
    r3 = jnp.concatenate([rt, rm, rb], axis=0).astype(jnp.float32)
    p3x = _mask_edge_rows((y3 + r3).astype(bf), first, last)
    o3_ref[...] = _conv3x3_valid(p3x, w32, b32_ref[...])[None]


def kernel(C3, C4, C5, p5_1_w, p5_1_b, p5_2_w, p5_2_b,
           p4_1_w, p4_1_b, p4_2_w, p4_2_b,
           p3_1_w, p3_1_b, p3_2_w, p3_2_b):
    N, H3, W3, C3c = C3.shape
    _, H4, W4, C4c = C4.shape
    _, H5, W5, C5c = C5.shape
    Ch = p5_1_w.shape[1]
    Co = p5_2_w.shape[-1]

    w52 = p5_2_w.reshape(3, 3 * Ch, Co)
    w42 = p4_2_w.reshape(3, 3 * Ch, Co)
    w32 = p3_2_w.reshape(3, 3 * Ch, Co)
    b5 = p5_1_b.reshape(1, Ch)
    b4 = p4_1_b.reshape(1, Ch)
    b3 = p3_1_b.reshape(1, Ch)
    b52 = p5_2_b.reshape(1, Co)
    b42 = p4_2_b.reshape(1, Co)
    b32 = p3_2_b.reshape(1, Co)

    res = lambda *blk: pl.BlockSpec(blk, lambda n, j: (0,) * len(blk))
    out3, out4, out5 = pl.pallas_call(
        _fused_kernel,
        out_shape=(
            jax.ShapeDtypeStruct((N, H3, W3, Co), jnp.float32),
            jax.ShapeDtypeStruct((N, H4, W4, Co), jnp.float32),
            jax.ShapeDtypeStruct((N, H5, W5, Co), jnp.float32),
        ),
        grid=(N, _J),
        in_specs=[
            pl.BlockSpec((1, H3, W3, C3c), lambda n, j: (n, 0, 0, 0)),
            pl.BlockSpec((1, H4, W4, C4c), lambda n, j: (n, 0, 0, 0)),
            pl.BlockSpec((1, H5, W5, C5c), lambda n, j: (n, 0, 0, 0)),
            res(C5c, Ch), res(1, Ch), res(3, 3 * Ch, Co), res(1, Co),
            res(C4c, Ch), res(1, Ch), res(3, 3 * Ch, Co), res(1, Co),
            res(C3c, Ch), res(1, Ch), res(3, 3 * Ch, Co), res(1, Co),
        ],
        out_specs=(
            pl.BlockSpec((1, H3 // _J, W3, Co), lambda n, j: (n, j, 0, 0)),
            pl.BlockSpec((1, H4 // _J, W4, Co), lambda n, j: (n, j, 0, 0)),
            pl.BlockSpec((1, H5 // _J, W5, Co), lambda n, j: (n, j, 0, 0)),
        ),
        scratch_shapes=[
            pltpu.VMEM((H5, W5, Ch), jnp.bfloat16),
            pltpu.VMEM((H4, W4, Ch), jnp.bfloat16),
        ],
        compiler_params=pltpu.CompilerParams(
            dimension_semantics=("parallel", "arbitrary"),
            vmem_limit_bytes=100 * 1024 * 1024),
    )(C3, C4, C5,
      p5_1_w, b5, w52, b52,
      p4_1_w, b4, w42, b42,
      p3_1_w, b3, w32, b32)
    return [out3, out4, out5]
